# Initial kernel scaffold; baseline (speedup 1.0000x reference)
import jax
import jax.numpy as jnp
from jax.experimental import pallas as pl


def kernel(card_ids, costs, rarities, types, upgrades, card_id_table, cost_table, rarity_table, type_table, upgrade_table, W, b):
    raise NotImplementedError("write your pallas kernel here")


# speedup vs baseline: 13.9869x; 13.9869x over previous
import functools

import jax
import jax.numpy as jnp
from jax import lax
from jax.experimental import pallas as pl
from jax.experimental.pallas import tpu as pltpu
from jax.experimental.pallas import tpu_sc as plsc

_B, _L = 4096, 200
_N = _B * _L
_D = 128
_NC_COST, _NC_RAR, _NC_TYP, _NC_UPG = 11, 5, 4, 21
_NSMALL = _NC_COST * _NC_RAR * _NC_TYP * _NC_UPG
_SUP = 1024
_CH = 128


def _prep_body(cid_ref, cost_ref, rar_ref, typ_ref, upg_ref, w_ref, b_ref,
               tid_ref, tsmall_ref):
    w = w_ref[...]
    half = cid_ref.shape[1]
    tid_ref[...] = jnp.dot(cid_ref[...], w[0:half, :],
                           preferred_element_type=jnp.float32)
    o0 = half
    fc = jnp.dot(cost_ref[...], w[o0:o0 + 8, :],
                 preferred_element_type=jnp.float32)
    fr = jnp.dot(rar_ref[...], w[o0 + 8:o0 + 16, :],
                 preferred_element_type=jnp.float32)
    ft = jnp.dot(typ_ref[...], w[o0 + 16:o0 + 24, :],
                 preferred_element_type=jnp.float32)
    fu = jnp.dot(upg_ref[...], w[o0 + 24:o0 + 32, :],
                 preferred_element_type=jnp.float32)
    k = lax.broadcasted_iota(jnp.int32, (_NSMALL, 1), 0)
    oc = (k // (_NC_RAR * _NC_TYP * _NC_UPG) ==
          lax.broadcasted_iota(jnp.int32, (_NSMALL, _NC_COST), 1)
          ).astype(jnp.float32)
    orr = ((k // (_NC_TYP * _NC_UPG)) % _NC_RAR ==
           lax.broadcasted_iota(jnp.int32, (_NSMALL, _NC_RAR), 1)
           ).astype(jnp.float32)
    ot = ((k // _NC_UPG) % _NC_TYP ==
          lax.broadcasted_iota(jnp.int32, (_NSMALL, _NC_TYP), 1)
          ).astype(jnp.float32)
    ou = (k % _NC_UPG ==
          lax.broadcasted_iota(jnp.int32, (_NSMALL, _NC_UPG), 1)
          ).astype(jnp.float32)
    tsmall_ref[...] = (jnp.dot(oc, fc, preferred_element_type=jnp.float32)
                       + jnp.dot(orr, fr, preferred_element_type=jnp.float32)
                       + jnp.dot(ot, ft, preferred_element_type=jnp.float32)
                       + jnp.dot(ou, fu, preferred_element_type=jnp.float32)
                       + b_ref[...])


def _sc_body(nc, per_w, nsup, nch,
             ids_h, c_h, r_h, t_h, u_h, tid_h, tsm_h, out_h,
             idv, cv, rv, tv, uv, combov, bufa, bufb, gsem):
    wid = lax.axis_index("s") * nc + lax.axis_index("c")
    base = wid * per_w

    def sup_body(si, carry):
        sb = base + si * _SUP
        pltpu.sync_copy(ids_h.at[pl.ds(sb, _SUP)], idv)
        pltpu.sync_copy(c_h.at[pl.ds(sb, _SUP)], cv)
        pltpu.sync_copy(r_h.at[pl.ds(sb, _SUP)], rv)
        pltpu.sync_copy(t_h.at[pl.ds(sb, _SUP)], tv)
        pltpu.sync_copy(u_h.at[pl.ds(sb, _SUP)], uv)

        m_r = _NC_TYP * _NC_UPG * _NC_RAR
        del m_r

        def combo_body(v, cc2):
            off = v * 16
            cc = cv[pl.ds(off, 16)]
            rr = rv[pl.ds(off, 16)]
            tt = tv[pl.ds(off, 16)]
            uu = uv[pl.ds(off, 16)]
            combov[pl.ds(off, 16)] = (cc * (_NC_RAR * _NC_TYP * _NC_UPG)
                                      + rr * (_NC_TYP * _NC_UPG)
                                      + tt * _NC_UPG + uu)
            return cc2
        lax.fori_loop(0, _SUP // 16, combo_body, 0)

        for j in range(nch):
            s = j % 2
            ca = pltpu.async_copy(tid_h.at[idv.at[pl.ds(j * _CH, _CH)]],
                                  bufa.at[s], gsem)
            cb = pltpu.async_copy(tsm_h.at[combov.at[pl.ds(j * _CH, _CH)]],
                                  bufb.at[s], gsem)
            ca.wait()
            cb.wait()

            def add_body(row, cc3, s=s):
                for col in range(_D // 16):
                    o = col * 16
                    bufa[s, row, pl.ds(o, 16)] = (bufa[s, row, pl.ds(o, 16)]
                                                  + bufb[s, row, pl.ds(o, 16)])
                return cc3
            lax.fori_loop(0, _CH, add_body, 0)
            pltpu.sync_copy(bufa.at[s],
                            out_h.at[pl.ds(sb + j * _CH, _CH), :])
        return carry

    lax.fori_loop(0, nsup, sup_body, 0)


def kernel(card_ids, costs, rarities, types, upgrades, card_id_table,
           cost_table, rarity_table, type_table, upgrade_table, W, b):
    nrows = card_id_table.shape[0]
    tid, tsm = pl.pallas_call(
        _prep_body,
        out_shape=[
            jax.ShapeDtypeStruct((nrows, _D), jnp.float32),
            jax.ShapeDtypeStruct((_NSMALL, _D), jnp.float32),
        ],
    )(card_id_table, cost_table, rarity_table, type_table, upgrade_table, W,
      b.reshape(1, _D))

    info = plsc.get_sparse_core_info()
    nc, ns = info.num_cores, info.num_subcores
    nw = nc * ns
    per_w = _N // nw
    assert _N % nw == 0 and per_w % _SUP == 0
    nsup = per_w // _SUP
    nch = _SUP // _CH

    mesh = plsc.VectorSubcoreMesh(core_axis_name="c", subcore_axis_name="s")
    sc = pl.kernel(
        functools.partial(_sc_body, nc, per_w, nsup, nch),
        out_type=jax.ShapeDtypeStruct((_N, _D), jnp.float32),
        mesh=mesh,
        scratch_types=[
            pltpu.VMEM((_SUP,), jnp.int32),
            pltpu.VMEM((_SUP,), jnp.int32),
            pltpu.VMEM((_SUP,), jnp.int32),
            pltpu.VMEM((_SUP,), jnp.int32),
            pltpu.VMEM((_SUP,), jnp.int32),
            pltpu.VMEM((_SUP,), jnp.int32),
            pltpu.VMEM((2, _CH, _D), jnp.float32),
            pltpu.VMEM((2, _CH, _D), jnp.float32),
            pltpu.SemaphoreType.DMA,
        ],
    )
    out = sc(card_ids.reshape(-1).astype(jnp.int32),
             costs.reshape(-1).astype(jnp.int32),
             rarities.reshape(-1).astype(jnp.int32),
             types.reshape(-1).astype(jnp.int32),
             upgrades.reshape(-1).astype(jnp.int32),
             tid, tsm)
    return out.reshape(_B, _L, _D)


# speedup vs baseline: 20.4654x; 1.4632x over previous
import functools

import jax
import jax.numpy as jnp
from jax import lax
from jax.experimental import pallas as pl
from jax.experimental.pallas import tpu as pltpu
from jax.experimental.pallas import tpu_sc as plsc

_B, _L = 4096, 200
_N = _B * _L
_D = 128
_NC_COST, _NC_RAR, _NC_TYP, _NC_UPG = 11, 5, 4, 21
_NSMALL = _NC_COST * _NC_RAR * _NC_TYP * _NC_UPG
_SUP = 3200
_CH = 128


def _prep_body(cid_ref, cost_ref, rar_ref, typ_ref, upg_ref, w_ref, b_ref,
               tid_ref, tsmall_ref):
    w = w_ref[...]
    half = cid_ref.shape[1]
    tid_ref[...] = jnp.dot(cid_ref[...], w[0:half, :],
                           preferred_element_type=jnp.float32)
    o0 = half
    fc = jnp.dot(cost_ref[...], w[o0:o0 + 8, :],
                 preferred_element_type=jnp.float32)
    fr = jnp.dot(rar_ref[...], w[o0 + 8:o0 + 16, :],
                 preferred_element_type=jnp.float32)
    ft = jnp.dot(typ_ref[...], w[o0 + 16:o0 + 24, :],
                 preferred_element_type=jnp.float32)
    fu = jnp.dot(upg_ref[...], w[o0 + 24:o0 + 32, :],
                 preferred_element_type=jnp.float32)
    k = lax.broadcasted_iota(jnp.int32, (_NSMALL, 1), 0)
    oc = (k // (_NC_RAR * _NC_TYP * _NC_UPG) ==
          lax.broadcasted_iota(jnp.int32, (_NSMALL, _NC_COST), 1)
          ).astype(jnp.float32)
    orr = ((k // (_NC_TYP * _NC_UPG)) % _NC_RAR ==
           lax.broadcasted_iota(jnp.int32, (_NSMALL, _NC_RAR), 1)
           ).astype(jnp.float32)
    ot = ((k // _NC_UPG) % _NC_TYP ==
          lax.broadcasted_iota(jnp.int32, (_NSMALL, _NC_TYP), 1)
          ).astype(jnp.float32)
    ou = (k % _NC_UPG ==
          lax.broadcasted_iota(jnp.int32, (_NSMALL, _NC_UPG), 1)
          ).astype(jnp.float32)
    tsmall_ref[...] = (jnp.dot(oc, fc, preferred_element_type=jnp.float32)
                       + jnp.dot(orr, fr, preferred_element_type=jnp.float32)
                       + jnp.dot(ot, ft, preferred_element_type=jnp.float32)
                       + jnp.dot(ou, fu, preferred_element_type=jnp.float32)
                       + b_ref[...])


def _sc_body(nc, per_w, nsup, nch,
             ids_h, c_h, r_h, t_h, u_h, tid_h, tsm_h, out_h,
             idv, cv, rv, tv, uv, combov, bufa, bufb,
             isem, gsem0, gsem1, ssem0, ssem1):
    wid = lax.axis_index("s") * nc + lax.axis_index("c")
    base = wid * per_w
    gsems = (gsem0, gsem1)
    ssems = (ssem0, ssem1)

    def sup_body(si, carry):
        sb = base + si * _SUP
        ih = [pltpu.async_copy(src.at[pl.ds(sb, _SUP)], dst, isem)
              for src, dst in ((ids_h, idv), (c_h, cv), (r_h, rv),
                               (t_h, tv), (u_h, uv))]
        for h in ih:
            h.wait()

        def combo_chunk(j):
            def combo_body(v, cc2, j=j):
                off = j * _CH + v * 16
                combov[pl.ds(off, 16)] = (
                    cv[pl.ds(off, 16)] * (_NC_RAR * _NC_TYP * _NC_UPG)
                    + rv[pl.ds(off, 16)] * (_NC_TYP * _NC_UPG)
                    + tv[pl.ds(off, 16)] * _NC_UPG
                    + uv[pl.ds(off, 16)])
                return cc2
            lax.fori_loop(0, _CH // 16, combo_body, 0)

        def issue(j):
            s = j % 2
            return (
                pltpu.async_copy(tid_h.at[idv.at[pl.ds(j * _CH, _CH)]],
                                 bufa.at[s], gsems[s]),
                pltpu.async_copy(tsm_h.at[combov.at[pl.ds(j * _CH, _CH)]],
                                 bufb.at[s], gsems[s]),
            )

        g = [None, None]
        st = [None, None]
        combo_chunk(0)
        g[0] = issue(0)
        for j in range(nch):
            s = j % 2
            sn = (j + 1) % 2
            if j + 1 < nch:
                combo_chunk(j + 1)
                if j >= 1:
                    st[sn].wait()
                g[sn] = issue(j + 1)
            g[s][0].wait()
            g[s][1].wait()

            def add_body(row, cc3, s=s):
                for col in range(_D // 16):
                    o = col * 16
                    bufa[s, row, pl.ds(o, 16)] = (bufa[s, row, pl.ds(o, 16)]
                                                  + bufb[s, row, pl.ds(o, 16)])
                return cc3
            lax.fori_loop(0, _CH, add_body, 0)
            st[s] = pltpu.async_copy(bufa.at[s],
                                     out_h.at[pl.ds(sb + j * _CH, _CH), :],
                                     ssems[s])
        st[0].wait()
        st[1].wait()
        return carry

    lax.fori_loop(0, nsup, sup_body, 0)


def kernel(card_ids, costs, rarities, types, upgrades, card_id_table,
           cost_table, rarity_table, type_table, upgrade_table, W, b):
    nrows = card_id_table.shape[0]
    tid, tsm = pl.pallas_call(
        _prep_body,
        out_shape=[
            jax.ShapeDtypeStruct((nrows, _D), jnp.float32),
            jax.ShapeDtypeStruct((_NSMALL, _D), jnp.float32),
        ],
    )(card_id_table, cost_table, rarity_table, type_table, upgrade_table, W,
      b.reshape(1, _D))

    info = plsc.get_sparse_core_info()
    nc, ns = info.num_cores, info.num_subcores
    nw = nc * ns
    per_w = _N // nw
    assert _N % nw == 0 and per_w % _SUP == 0
    nsup = per_w // _SUP
    nch = _SUP // _CH

    mesh = plsc.VectorSubcoreMesh(core_axis_name="c", subcore_axis_name="s")
    sc = pl.kernel(
        functools.partial(_sc_body, nc, per_w, nsup, nch),
        out_type=jax.ShapeDtypeStruct((_N, _D), jnp.float32),
        mesh=mesh,
        scratch_types=[
            pltpu.VMEM((_SUP,), jnp.int32),
            pltpu.VMEM((_SUP,), jnp.int32),
            pltpu.VMEM((_SUP,), jnp.int32),
            pltpu.VMEM((_SUP,), jnp.int32),
            pltpu.VMEM((_SUP,), jnp.int32),
            pltpu.VMEM((_SUP,), jnp.int32),
            pltpu.VMEM((2, _CH, _D), jnp.float32),
            pltpu.VMEM((2, _CH, _D), jnp.float32),
            pltpu.SemaphoreType.DMA,
            pltpu.SemaphoreType.DMA,
            pltpu.SemaphoreType.DMA,
            pltpu.SemaphoreType.DMA,
            pltpu.SemaphoreType.DMA,
        ],
    )
    out = sc(card_ids.reshape(-1).astype(jnp.int32),
             costs.reshape(-1).astype(jnp.int32),
             rarities.reshape(-1).astype(jnp.int32),
             types.reshape(-1).astype(jnp.int32),
             upgrades.reshape(-1).astype(jnp.int32),
             tid, tsm)
    return out.reshape(_B, _L, _D)


# speedup vs baseline: 20.5234x; 1.0028x over previous
import functools

import jax
import jax.numpy as jnp
from jax import lax
from jax.experimental import pallas as pl
from jax.experimental.pallas import tpu as pltpu
from jax.experimental.pallas import tpu_sc as plsc

_B, _L = 4096, 200
_N = _B * _L
_D = 128
_NC_COST, _NC_RAR, _NC_TYP, _NC_UPG = 11, 5, 4, 21
_NSMALL = _NC_COST * _NC_RAR * _NC_TYP * _NC_UPG
_SUP = 1600
_CH = 64
_NSLOT = 4


def _prep_body(cid_ref, cost_ref, rar_ref, typ_ref, upg_ref, w_ref, b_ref,
               tid_ref, tsmall_ref):
    w = w_ref[...]
    half = cid_ref.shape[1]
    tid_ref[...] = jnp.dot(cid_ref[...], w[0:half, :],
                           preferred_element_type=jnp.float32)
    o0 = half
    fc = jnp.dot(cost_ref[...], w[o0:o0 + 8, :],
                 preferred_element_type=jnp.float32)
    fr = jnp.dot(rar_ref[...], w[o0 + 8:o0 + 16, :],
                 preferred_element_type=jnp.float32)
    ft = jnp.dot(typ_ref[...], w[o0 + 16:o0 + 24, :],
                 preferred_element_type=jnp.float32)
    fu = jnp.dot(upg_ref[...], w[o0 + 24:o0 + 32, :],
                 preferred_element_type=jnp.float32)
    k = lax.broadcasted_iota(jnp.int32, (_NSMALL, 1), 0)
    oc = (k // (_NC_RAR * _NC_TYP * _NC_UPG) ==
          lax.broadcasted_iota(jnp.int32, (_NSMALL, _NC_COST), 1)
          ).astype(jnp.float32)
    orr = ((k // (_NC_TYP * _NC_UPG)) % _NC_RAR ==
           lax.broadcasted_iota(jnp.int32, (_NSMALL, _NC_RAR), 1)
           ).astype(jnp.float32)
    ot = ((k // _NC_UPG) % _NC_TYP ==
          lax.broadcasted_iota(jnp.int32, (_NSMALL, _NC_TYP), 1)
          ).astype(jnp.float32)
    ou = (k % _NC_UPG ==
          lax.broadcasted_iota(jnp.int32, (_NSMALL, _NC_UPG), 1)
          ).astype(jnp.float32)
    tsmall_ref[...] = (jnp.dot(oc, fc, preferred_element_type=jnp.float32)
                       + jnp.dot(orr, fr, preferred_element_type=jnp.float32)
                       + jnp.dot(ot, ft, preferred_element_type=jnp.float32)
                       + jnp.dot(ou, fu, preferred_element_type=jnp.float32)
                       + b_ref[...])


def _sc_body(nc, per_w, nsup, nch,
             ids_h, c_h, r_h, t_h, u_h, tid_h, tsm_h, out_h,
             idv, cv, rv, tv, uv, combov, bufa, bufb,
             isem, gsem0, gsem1, gsem2, gsem3,
             ssem0, ssem1, ssem2, ssem3):
    wid = lax.axis_index("s") * nc + lax.axis_index("c")
    base = wid * per_w
    gsems = (gsem0, gsem1, gsem2, gsem3)
    ssems = (ssem0, ssem1, ssem2, ssem3)
    depth = _NSLOT - 1

    def sup_body(si, carry):
        sb = base + si * _SUP
        ih = [pltpu.async_copy(src.at[pl.ds(sb, _SUP)], dst, isem)
              for src, dst in ((ids_h, idv), (c_h, cv), (r_h, rv),
                               (t_h, tv), (u_h, uv))]
        for h in ih:
            h.wait()

        def combo_chunk(j):
            def combo_body(v, cc2, j=j):
                off = j * _CH + v * 16
                combov[pl.ds(off, 16)] = (
                    cv[pl.ds(off, 16)] * (_NC_RAR * _NC_TYP * _NC_UPG)
                    + rv[pl.ds(off, 16)] * (_NC_TYP * _NC_UPG)
                    + tv[pl.ds(off, 16)] * _NC_UPG
                    + uv[pl.ds(off, 16)])
                return cc2
            lax.fori_loop(0, _CH // 16, combo_body, 0)

        def issue(j):
            s = j % _NSLOT
            return (
                pltpu.async_copy(tid_h.at[idv.at[pl.ds(j * _CH, _CH)]],
                                 bufa.at[s], gsems[s]),
                pltpu.async_copy(tsm_h.at[combov.at[pl.ds(j * _CH, _CH)]],
                                 bufb.at[s], gsems[s]),
            )

        g = [None] * _NSLOT
        st = [None] * _NSLOT
        for k in range(min(depth, nch)):
            combo_chunk(k)
            g[k] = issue(k)
        for j in range(nch):
            s = j % _NSLOT
            jn = j + depth
            if jn < nch:
                sn = jn % _NSLOT
                combo_chunk(jn)
                if st[sn] is not None:
                    st[sn].wait()
                g[sn] = issue(jn)
            g[s][0].wait()
            g[s][1].wait()

            def add_body(row, cc3, s=s):
                for col in range(_D // 16):
                    o = col * 16
                    plsc.addupdate(bufa.at[s, row, pl.ds(o, 16)],
                                   bufb[s, row, pl.ds(o, 16)])
                return cc3
            lax.fori_loop(0, _CH, add_body, 0)
            st[s] = pltpu.async_copy(bufa.at[s],
                                     out_h.at[pl.ds(sb + j * _CH, _CH), :],
                                     ssems[s])
        for s in range(_NSLOT):
            if st[s] is not None:
                st[s].wait()
        return carry

    lax.fori_loop(0, nsup, sup_body, 0)


def kernel(card_ids, costs, rarities, types, upgrades, card_id_table,
           cost_table, rarity_table, type_table, upgrade_table, W, b):
    nrows = card_id_table.shape[0]
    tid, tsm = pl.pallas_call(
        _prep_body,
        out_shape=[
            jax.ShapeDtypeStruct((nrows, _D), jnp.float32),
            jax.ShapeDtypeStruct((_NSMALL, _D), jnp.float32),
        ],
    )(card_id_table, cost_table, rarity_table, type_table, upgrade_table, W,
      b.reshape(1, _D))

    info = plsc.get_sparse_core_info()
    nc, ns = info.num_cores, info.num_subcores
    nw = nc * ns
    per_w = _N // nw
    assert _N % nw == 0 and per_w % _SUP == 0
    nsup = per_w // _SUP
    nch = _SUP // _CH

    mesh = plsc.VectorSubcoreMesh(core_axis_name="c", subcore_axis_name="s")
    sc = pl.kernel(
        functools.partial(_sc_body, nc, per_w, nsup, nch),
        out_type=jax.ShapeDtypeStruct((_N, _D), jnp.float32),
        mesh=mesh,
        scratch_types=[
            pltpu.VMEM((_SUP,), jnp.int32),
            pltpu.VMEM((_SUP,), jnp.int32),
            pltpu.VMEM((_SUP,), jnp.int32),
            pltpu.VMEM((_SUP,), jnp.int32),
            pltpu.VMEM((_SUP,), jnp.int32),
            pltpu.VMEM((_SUP,), jnp.int32),
            pltpu.VMEM((_NSLOT, _CH, _D), jnp.float32),
            pltpu.VMEM((_NSLOT, _CH, _D), jnp.float32),
            pltpu.SemaphoreType.DMA,
            pltpu.SemaphoreType.DMA,
            pltpu.SemaphoreType.DMA,
            pltpu.SemaphoreType.DMA,
            pltpu.SemaphoreType.DMA,
            pltpu.SemaphoreType.DMA,
            pltpu.SemaphoreType.DMA,
            pltpu.SemaphoreType.DMA,
            pltpu.SemaphoreType.DMA,
        ],
    )
    out = sc(card_ids.reshape(-1).astype(jnp.int32),
             costs.reshape(-1).astype(jnp.int32),
             rarities.reshape(-1).astype(jnp.int32),
             types.reshape(-1).astype(jnp.int32),
             upgrades.reshape(-1).astype(jnp.int32),
             tid, tsm)
    return out.reshape(_B, _L, _D)
